# baseline (speedup 1.0000x reference)
import functools

import jax
import jax.numpy as jnp
from jax import lax
from jax.experimental import pallas as pl
from jax.experimental.pallas import tpu as pltpu
from jax.experimental.pallas import tpu_sc as plsc

N = 10000
E = 320000
D = 128

NC = 2
NS = 16
NW = NC * NS
EPW = E // NW
CH = 80
NCHUNK = EPW // CH
RPT = 640
ZR = 128

_f32 = jnp.float32
_i32 = jnp.int32


def _mesh():
    return plsc.VectorSubcoreMesh(
        core_axis_name="c", subcore_axis_name="s", num_cores=NC, num_subcores=NS
    )


DEG_ROWS = E // 128
DEG_FULL = DEG_ROWS // 8


def _sc_degree(dst2, ew2):
    @functools.partial(
        pl.kernel,
        out_type=jax.ShapeDtypeStruct((NC * N,), _f32),
        mesh=_mesh(),
        scratch_types=[
            pltpu.VMEM((8, 128), _i32),
            pltpu.VMEM((8, 128), _f32),
            pltpu.VMEM((640,), _f32),
            pltpu.VMEM_SHARED((N,), _f32),
            pltpu.SemaphoreType.DMA,
        ],
    )
    def deg_kernel(dst_hbm, ew_hbm, out_hbm, dstb, ewb, zb, acc, sem):
        c = lax.axis_index("c")
        s = lax.axis_index("s")
        wid = s * NC + c
        zeros16 = jnp.zeros((16,), _f32)
        for i in range(40):
            zb[pl.ds(i * 16, 16)] = zeros16
        @pl.when(s < 15)
        def _():
            pltpu.sync_copy(zb, acc.at[pl.ds(s * 640, 640)])

        @pl.when(s == 15)
        def _():
            pltpu.sync_copy(zb.at[pl.ds(0, 400)], acc.at[pl.ds(9600, 400)])

        plsc.subcore_barrier()

        def do_block(m, nrow):
            pltpu.sync_copy(dst_hbm.at[pl.ds(8 * m, nrow)],
                            dstb.at[pl.ds(0, nrow)])
            pltpu.sync_copy(ew_hbm.at[pl.ds(8 * m, nrow)],
                            ewb.at[pl.ds(0, nrow)])
            descs = [pltpu.async_copy(ewb.at[j], acc.at[dstb.at[j]], sem,
                                      add=True) for j in range(nrow)]
            for dsc in descs:
                dsc.wait()

        def rnd(t, carry):
            do_block(wid + NW * t, 8)
            return carry

        lax.fori_loop(0, DEG_FULL // NW, rnd, 0)

        @pl.when(wid < DEG_FULL - (DEG_FULL // NW) * NW)
        def _():
            do_block((DEG_FULL // NW) * NW + wid, 8)

        @pl.when(wid == NW - 1)
        def _():
            do_block(DEG_FULL, DEG_ROWS - DEG_FULL * 8)

        plsc.subcore_barrier()

        @pl.when(s < 15)
        def _():
            pltpu.sync_copy(acc.at[pl.ds(s * 640, 640)], zb)
            pltpu.sync_copy(zb, out_hbm.at[pl.ds(c * N + s * 640, 640)])

        @pl.when(s == 15)
        def _():
            pltpu.sync_copy(acc.at[pl.ds(9600, 400)], zb.at[pl.ds(0, 400)])
            pltpu.sync_copy(zb.at[pl.ds(0, 400)],
                            out_hbm.at[pl.ds(c * N + 9600, 400)])

    return deg_kernel(dst2, ew2)


ECH = 128
NCH_TOT = E // ECH
FULL_T = NCH_TOT // NW


def _sc_aggregate(hp, ei, ewr):
    @functools.partial(
        pl.kernel,
        out_type=jax.ShapeDtypeStruct((NC * N, D), _f32),
        mesh=_mesh(),
        scratch_types=[
            pltpu.VMEM((2, 2, ECH), _i32),
            pltpu.VMEM((2, ECH // 8, D), _f32),
            pltpu.VMEM((2, ECH, D), _f32),
            pltpu.VMEM_SHARED((N, D), _f32),
            pltpu.SemaphoreType.DMA,
            pltpu.SemaphoreType.DMA,
            pltpu.SemaphoreType.DMA,
        ],
    )
    def agg_kernel(hp_hbm, ei_hbm, ewr_hbm, out_hbm,
                   sd_v, ewr_v, rows_v, acc, sem_g, sem_s, sem_i):
        c = lax.axis_index("c")
        s = lax.axis_index("s")
        wid = s * NC + c
        zeros16 = jnp.zeros((16,), _f32)

        def zrow(i, carry):
            for j in range(D // 16):
                rows_v[0, i, pl.ds(j * 16, 16)] = zeros16
            return carry

        lax.fori_loop(0, ZR, zrow, 0)

        @pl.when(s < 15)
        def _():
            for k in range(RPT // ZR):
                pltpu.sync_copy(rows_v.at[0], acc.at[pl.ds(s * RPT + k * ZR, ZR)])

        @pl.when(s == 15)
        def _():
            for k in range(3):
                pltpu.sync_copy(rows_v.at[0], acc.at[pl.ds(9600 + k * ZR, ZR)])
            pltpu.sync_copy(rows_v.at[0, pl.ds(0, 16)], acc.at[pl.ds(9984, 16)])

        plsc.subcore_barrier()

        n_tail = NCH_TOT - FULL_T * NW
        t_w = FULL_T + jnp.where(wid < n_tail, 1, 0)

        def cidx(t):
            return wid + NW * t

        def issue_idx(t, q):
            pltpu.async_copy(ei_hbm.at[:, pl.ds(cidx(t) * ECH, ECH)],
                             sd_v.at[q], sem_i)
            pltpu.async_copy(
                ewr_hbm.at[pl.ds(cidx(t) * (ECH // 8), ECH // 8)],
                ewr_v.at[q], sem_i)

        def wait_idx(t, q):
            pltpu.make_async_copy(ei_hbm.at[:, pl.ds(cidx(t) * ECH, ECH)],
                                  sd_v.at[q], sem_i).wait()
            pltpu.make_async_copy(
                ewr_hbm.at[pl.ds(cidx(t) * (ECH // 8), ECH // 8)],
                ewr_v.at[q], sem_i).wait()

        def issue_gather(q, b):
            pltpu.async_copy(hp_hbm.at[sd_v.at[q, 0]], rows_v.at[b], sem_g)

        def wait_gather(q, b):
            pltpu.make_async_copy(hp_hbm.at[sd_v.at[q, 0]], rows_v.at[b],
                                  sem_g).wait()

        def issue_scatter(q, b):
            pltpu.async_copy(rows_v.at[b], acc.at[sd_v.at[q, 1]], sem_s,
                             add=True)

        def wait_scatter(q, b):
            pltpu.make_async_copy(rows_v.at[b], acc.at[sd_v.at[q, 1]],
                                  sem_s).wait()

        def scale(q, b):
            for r in range(ECH // 8):
                for l in range(8):
                    w = ewr_v[q, r, pl.ds(l * 16, 16)]
                    e = r * 8 + l
                    for j in range(D // 16):
                        rows_v[b, e, pl.ds(j * 16, 16)] = (
                            rows_v[b, e, pl.ds(j * 16, 16)] * w)

        issue_idx(0, 0)
        wait_idx(0, 0)
        issue_gather(0, 0)

        def body(t, carry):
            b = t & 1
            b2 = 1 - b

            @pl.when(t + 1 < t_w)
            def _():
                @pl.when(t >= 1)
                def _():
                    wait_scatter(b2, b2)

                issue_idx(t + 1, b2)
                wait_idx(t + 1, b2)
                issue_gather(b2, b2)

            wait_gather(b, b)
            scale(b, b)
            issue_scatter(b, b)
            return carry

        lax.fori_loop(0, t_w, body, 0)
        wait_scatter(0, 0)
        wait_scatter(0, 1)
        plsc.subcore_barrier()

        @pl.when(s < 15)
        def _():
            for k in range(RPT // ZR):
                pltpu.sync_copy(acc.at[pl.ds(s * RPT + k * ZR, ZR)], rows_v.at[0])
                pltpu.sync_copy(rows_v.at[0],
                                out_hbm.at[pl.ds(c * N + s * RPT + k * ZR, ZR)])

        @pl.when(s == 15)
        def _():
            for k in range(3):
                pltpu.sync_copy(acc.at[pl.ds(9600 + k * ZR, ZR)], rows_v.at[0])
                pltpu.sync_copy(rows_v.at[0],
                                out_hbm.at[pl.ds(c * N + 9600 + k * ZR, ZR)])
            pltpu.sync_copy(acc.at[pl.ds(9984, 16)], rows_v.at[0, pl.ds(0, 16)])
            pltpu.sync_copy(rows_v.at[0, pl.ds(0, 16)],
                            out_hbm.at[pl.ds(c * N + 9984, 16)])

    return agg_kernel(hp, ei, ewr)


def _tc_pre(x, W1, deg2, ew2):
    def body(x_ref, w_ref, deg_ref, ew_ref, hp_ref, dis_ref, ewr_ref):
        deg = deg_ref[0] + deg_ref[1] + 1.0
        dis = jnp.where(deg > 0, lax.rsqrt(jnp.maximum(deg, 1e-12)), 0.0)
        h = jnp.dot(x_ref[...], w_ref[...], preferred_element_type=_f32)
        hp_ref[...] = h * dis[:, None]
        dis_ref[...] = dis[:, None]
        col = lax.broadcasted_iota(_i32, (8, D), 1) // 16
        row = lax.broadcasted_iota(_i32, (8, D), 0)
        rep = jnp.where(col == row, 1.0, 0.0).astype(_f32)
        ewr_ref[...] = jnp.dot(ew_ref[...], rep, preferred_element_type=_f32)

    return pl.pallas_call(
        body,
        out_shape=(
            jax.ShapeDtypeStruct((N, D), _f32),
            jax.ShapeDtypeStruct((N, 1), _f32),
            jax.ShapeDtypeStruct((E // 8, D), _f32),
        ),
    )(x, W1, deg2, ew2)


def _tc_mid(s2, hp, dis, b, Wn):
    def body(s_ref, hp_ref, dis_ref, b_ref, w_ref, out_ref):
        t = (s_ref[0] + s_ref[1] + hp_ref[...]) * dis_ref[...] + b_ref[...]
        a = jnp.maximum(t, 0.0)
        h = jnp.dot(a, w_ref[...], preferred_element_type=_f32)
        out_ref[...] = h * dis_ref[...]

    return pl.pallas_call(
        body,
        out_shape=jax.ShapeDtypeStruct((N, D), _f32),
    )(s2, hp, dis, b, Wn)


def _tc_fin(s2, hp, dis, b):
    def body(s_ref, hp_ref, dis_ref, b_ref, out_ref):
        out_ref[...] = (s_ref[0] + s_ref[1] + hp_ref[...]) * dis_ref[...] + b_ref[...]

    return pl.pallas_call(
        body,
        out_shape=jax.ShapeDtypeStruct((N, D), _f32),
    )(s2, hp, dis, b)


def kernel(x, edge_index, edge_weight, W1, b1, W2, b2, W3, b3):
    ei = edge_index
    ew = edge_weight.astype(_f32)

    deg2 = _sc_degree(ei[1].reshape(DEG_ROWS, 128),
                      ew.reshape(DEG_ROWS, 128)).reshape(NC, N)
    hp1, dis, ewr = _tc_pre(x, W1, deg2, ew.reshape(E // 8, 8))
    s1 = _sc_aggregate(hp1, ei, ewr).reshape(NC, N, D)
    hp2 = _tc_mid(s1, hp1, dis, b1, W2)
    s2 = _sc_aggregate(hp2, ei, ewr).reshape(NC, N, D)
    hp3 = _tc_mid(s2, hp2, dis, b2, W3)
    s3 = _sc_aggregate(hp3, ei, ewr).reshape(NC, N, D)
    return _tc_fin(s3, hp3, dis, b3)


# speedup vs baseline: 1.5365x; 1.5365x over previous
import functools

import jax
import jax.numpy as jnp
from jax import lax
from jax.experimental import pallas as pl
from jax.experimental.pallas import tpu as pltpu
from jax.experimental.pallas import tpu_sc as plsc

N = 10000
E = 320000
D = 128

NC = 2
NS = 16
NW = NC * NS
EPW = E // NW
CH = 80
NCHUNK = EPW // CH
RPT = 640
ZR = 128

_f32 = jnp.float32
_i32 = jnp.int32


def _mesh():
    return plsc.VectorSubcoreMesh(
        core_axis_name="c", subcore_axis_name="s", num_cores=NC, num_subcores=NS
    )


DEG_ROWS = E // 128
DEG_FULL = DEG_ROWS // 8


def _sc_degree(dst2, ew2):
    @functools.partial(
        pl.kernel,
        out_type=jax.ShapeDtypeStruct((NC * N,), _f32),
        mesh=_mesh(),
        scratch_types=[
            pltpu.VMEM((8, 128), _i32),
            pltpu.VMEM((8, 128), _f32),
            pltpu.VMEM((640,), _f32),
            pltpu.VMEM_SHARED((N,), _f32),
            pltpu.SemaphoreType.DMA,
        ],
    )
    def deg_kernel(dst_hbm, ew_hbm, out_hbm, dstb, ewb, zb, acc, sem):
        c = lax.axis_index("c")
        s = lax.axis_index("s")
        wid = s * NC + c
        zeros16 = jnp.zeros((16,), _f32)
        for i in range(40):
            zb[pl.ds(i * 16, 16)] = zeros16
        @pl.when(s < 15)
        def _():
            pltpu.sync_copy(zb, acc.at[pl.ds(s * 640, 640)])

        @pl.when(s == 15)
        def _():
            pltpu.sync_copy(zb.at[pl.ds(0, 400)], acc.at[pl.ds(9600, 400)])

        plsc.subcore_barrier()

        def do_block(m, nrow):
            pltpu.sync_copy(dst_hbm.at[pl.ds(8 * m, nrow)],
                            dstb.at[pl.ds(0, nrow)])
            pltpu.sync_copy(ew_hbm.at[pl.ds(8 * m, nrow)],
                            ewb.at[pl.ds(0, nrow)])
            descs = [pltpu.async_copy(ewb.at[j], acc.at[dstb.at[j]], sem,
                                      add=True) for j in range(nrow)]
            for dsc in descs:
                dsc.wait()

        def rnd(t, carry):
            do_block(wid + NW * t, 8)
            return carry

        lax.fori_loop(0, DEG_FULL // NW, rnd, 0)

        @pl.when(wid < DEG_FULL - (DEG_FULL // NW) * NW)
        def _():
            do_block((DEG_FULL // NW) * NW + wid, 8)

        @pl.when(wid == NW - 1)
        def _():
            do_block(DEG_FULL, DEG_ROWS - DEG_FULL * 8)

        plsc.subcore_barrier()

        @pl.when(s < 15)
        def _():
            pltpu.sync_copy(acc.at[pl.ds(s * 640, 640)], zb)
            pltpu.sync_copy(zb, out_hbm.at[pl.ds(c * N + s * 640, 640)])

        @pl.when(s == 15)
        def _():
            pltpu.sync_copy(acc.at[pl.ds(9600, 400)], zb.at[pl.ds(0, 400)])
            pltpu.sync_copy(zb.at[pl.ds(0, 400)],
                            out_hbm.at[pl.ds(c * N + 9600, 400)])

    return deg_kernel(dst2, ew2)


ECH = 128
NCH_TOT = E // ECH
FULL_T = NCH_TOT // NW


def _sc_aggregate(hp, ei, ewr):
    @functools.partial(
        pl.kernel,
        out_type=jax.ShapeDtypeStruct((NC * N, D), _f32),
        mesh=_mesh(),
        scratch_types=[
            pltpu.VMEM((2, 2, ECH), _i32),
            pltpu.VMEM((2, ECH // 8, D), _f32),
            pltpu.VMEM((2, ECH, D), _f32),
            pltpu.VMEM_SHARED((N, D), _f32),
            pltpu.SemaphoreType.DMA,
            pltpu.SemaphoreType.DMA,
            pltpu.SemaphoreType.DMA,
        ],
    )
    def agg_kernel(hp_hbm, ei_hbm, ewr_hbm, out_hbm,
                   sd_v, ewr_v, rows_v, acc, sem_g, sem_s, sem_i):
        c = lax.axis_index("c")
        s = lax.axis_index("s")
        wid = s * NC + c
        zeros16 = jnp.zeros((16,), _f32)

        def zrow(i, carry):
            for j in range(D // 16):
                rows_v[0, i, pl.ds(j * 16, 16)] = zeros16
            return carry

        lax.fori_loop(0, ZR, zrow, 0)

        @pl.when(s < 15)
        def _():
            for k in range(RPT // ZR):
                pltpu.sync_copy(rows_v.at[0], acc.at[pl.ds(s * RPT + k * ZR, ZR)])

        @pl.when(s == 15)
        def _():
            for k in range(3):
                pltpu.sync_copy(rows_v.at[0], acc.at[pl.ds(9600 + k * ZR, ZR)])
            pltpu.sync_copy(rows_v.at[0, pl.ds(0, 16)], acc.at[pl.ds(9984, 16)])

        plsc.subcore_barrier()

        n_tail = NCH_TOT - FULL_T * NW
        t_w = FULL_T + jnp.where(wid < n_tail, 1, 0)

        def cidx(t):
            return wid + NW * t

        def issue_idx(t, q):
            pltpu.async_copy(ei_hbm.at[:, pl.ds(cidx(t) * ECH, ECH)],
                             sd_v.at[q], sem_i)
            pltpu.async_copy(
                ewr_hbm.at[pl.ds(cidx(t) * (ECH // 8), ECH // 8)],
                ewr_v.at[q], sem_i)

        def wait_idx(t, q):
            pltpu.make_async_copy(ei_hbm.at[:, pl.ds(cidx(t) * ECH, ECH)],
                                  sd_v.at[q], sem_i).wait()
            pltpu.make_async_copy(
                ewr_hbm.at[pl.ds(cidx(t) * (ECH // 8), ECH // 8)],
                ewr_v.at[q], sem_i).wait()

        def issue_gather(q, b):
            pltpu.async_copy(hp_hbm.at[sd_v.at[q, 0]], rows_v.at[b], sem_g)

        def wait_gather(q, b):
            pltpu.make_async_copy(hp_hbm.at[sd_v.at[q, 0]], rows_v.at[b],
                                  sem_g).wait()

        def issue_scatter(q, b):
            pltpu.async_copy(rows_v.at[b], acc.at[sd_v.at[q, 1]], sem_s,
                             add=True)

        def wait_scatter(q, b):
            pltpu.make_async_copy(rows_v.at[b], acc.at[sd_v.at[q, 1]],
                                  sem_s).wait()

        def scale(q, b):
            def scale_row(r, carry2):
                for l in range(8):
                    w = ewr_v[q, r, pl.ds(l * 16, 16)]
                    e = r * 8 + l
                    for j in range(D // 16):
                        rows_v[b, e, pl.ds(j * 16, 16)] = (
                            rows_v[b, e, pl.ds(j * 16, 16)] * w)
                return carry2

            lax.fori_loop(0, ECH // 8, scale_row, 0)

        issue_idx(0, 0)
        wait_idx(0, 0)
        issue_gather(0, 0)

        def body(t, carry):
            b = t & 1
            b2 = 1 - b

            @pl.when(t + 1 < t_w)
            def _():
                @pl.when(t >= 1)
                def _():
                    wait_scatter(b2, b2)

                issue_idx(t + 1, b2)
                wait_idx(t + 1, b2)
                issue_gather(b2, b2)

            wait_gather(b, b)
            scale(b, b)
            issue_scatter(b, b)
            return carry

        lax.fori_loop(0, t_w, body, 0)
        wait_scatter(0, 0)
        wait_scatter(0, 1)
        plsc.subcore_barrier()

        @pl.when(s < 15)
        def _():
            for k in range(RPT // ZR):
                pltpu.sync_copy(acc.at[pl.ds(s * RPT + k * ZR, ZR)], rows_v.at[0])
                pltpu.sync_copy(rows_v.at[0],
                                out_hbm.at[pl.ds(c * N + s * RPT + k * ZR, ZR)])

        @pl.when(s == 15)
        def _():
            for k in range(3):
                pltpu.sync_copy(acc.at[pl.ds(9600 + k * ZR, ZR)], rows_v.at[0])
                pltpu.sync_copy(rows_v.at[0],
                                out_hbm.at[pl.ds(c * N + 9600 + k * ZR, ZR)])
            pltpu.sync_copy(acc.at[pl.ds(9984, 16)], rows_v.at[0, pl.ds(0, 16)])
            pltpu.sync_copy(rows_v.at[0, pl.ds(0, 16)],
                            out_hbm.at[pl.ds(c * N + 9984, 16)])

    return agg_kernel(hp, ei, ewr)


def _tc_pre(x, W1, deg2, ew2):
    def body(x_ref, w_ref, deg_ref, ew_ref, hp_ref, dis_ref, ewr_ref):
        deg = deg_ref[0] + deg_ref[1] + 1.0
        dis = jnp.where(deg > 0, lax.rsqrt(jnp.maximum(deg, 1e-12)), 0.0)
        h = jnp.dot(x_ref[...], w_ref[...], preferred_element_type=_f32)
        hp_ref[...] = h * dis[:, None]
        dis_ref[...] = dis[:, None]
        col = lax.broadcasted_iota(_i32, (8, D), 1) // 16
        row = lax.broadcasted_iota(_i32, (8, D), 0)
        rep = jnp.where(col == row, 1.0, 0.0).astype(_f32)
        ewr_ref[...] = jnp.dot(ew_ref[...], rep, preferred_element_type=_f32)

    return pl.pallas_call(
        body,
        out_shape=(
            jax.ShapeDtypeStruct((N, D), _f32),
            jax.ShapeDtypeStruct((N, 1), _f32),
            jax.ShapeDtypeStruct((E // 8, D), _f32),
        ),
    )(x, W1, deg2, ew2)


def _tc_mid(s2, hp, dis, b, Wn):
    def body(s_ref, hp_ref, dis_ref, b_ref, w_ref, out_ref):
        t = (s_ref[0] + s_ref[1] + hp_ref[...]) * dis_ref[...] + b_ref[...]
        a = jnp.maximum(t, 0.0)
        h = jnp.dot(a, w_ref[...], preferred_element_type=_f32)
        out_ref[...] = h * dis_ref[...]

    return pl.pallas_call(
        body,
        out_shape=jax.ShapeDtypeStruct((N, D), _f32),
    )(s2, hp, dis, b, Wn)


def _tc_fin(s2, hp, dis, b):
    def body(s_ref, hp_ref, dis_ref, b_ref, out_ref):
        out_ref[...] = (s_ref[0] + s_ref[1] + hp_ref[...]) * dis_ref[...] + b_ref[...]

    return pl.pallas_call(
        body,
        out_shape=jax.ShapeDtypeStruct((N, D), _f32),
    )(s2, hp, dis, b)


def kernel(x, edge_index, edge_weight, W1, b1, W2, b2, W3, b3):
    ei = edge_index
    ew = edge_weight.astype(_f32)

    deg2 = _sc_degree(ei[1].reshape(DEG_ROWS, 128),
                      ew.reshape(DEG_ROWS, 128)).reshape(NC, N)
    hp1, dis, ewr = _tc_pre(x, W1, deg2, ew.reshape(E // 8, 8))
    s1 = _sc_aggregate(hp1, ei, ewr).reshape(NC, N, D)
    hp2 = _tc_mid(s1, hp1, dis, b1, W2)
    s2 = _sc_aggregate(hp2, ei, ewr).reshape(NC, N, D)
    hp3 = _tc_mid(s2, hp2, dis, b2, W3)
    s3 = _sc_aggregate(hp3, ei, ewr).reshape(NC, N, D)
    return _tc_fin(s3, hp3, dis, b3)


# speedup vs baseline: 1.7528x; 1.1408x over previous
import functools

import jax
import jax.numpy as jnp
from jax import lax
from jax.experimental import pallas as pl
from jax.experimental.pallas import tpu as pltpu
from jax.experimental.pallas import tpu_sc as plsc

N = 10000
E = 320000
D = 128

NC = 2
NS = 16
NW = NC * NS
EPW = E // NW
CH = 80
NCHUNK = EPW // CH
RPT = 640
ZR = 128

_f32 = jnp.float32
_i32 = jnp.int32


def _mesh():
    return plsc.VectorSubcoreMesh(
        core_axis_name="c", subcore_axis_name="s", num_cores=NC, num_subcores=NS
    )


DEG_ROWS = E // 128
DEG_FULL = DEG_ROWS // 8


def _sc_degree(dst2, ew2):
    @functools.partial(
        pl.kernel,
        out_type=jax.ShapeDtypeStruct((NC * N,), _f32),
        mesh=_mesh(),
        scratch_types=[
            pltpu.VMEM((8, 128), _i32),
            pltpu.VMEM((8, 128), _f32),
            pltpu.VMEM((640,), _f32),
            pltpu.VMEM_SHARED((N,), _f32),
            pltpu.SemaphoreType.DMA,
        ],
    )
    def deg_kernel(dst_hbm, ew_hbm, out_hbm, dstb, ewb, zb, acc, sem):
        c = lax.axis_index("c")
        s = lax.axis_index("s")
        wid = s * NC + c
        zeros16 = jnp.zeros((16,), _f32)
        for i in range(40):
            zb[pl.ds(i * 16, 16)] = zeros16
        @pl.when(s < 15)
        def _():
            pltpu.sync_copy(zb, acc.at[pl.ds(s * 640, 640)])

        @pl.when(s == 15)
        def _():
            pltpu.sync_copy(zb.at[pl.ds(0, 400)], acc.at[pl.ds(9600, 400)])

        plsc.subcore_barrier()

        def do_block(m, nrow):
            pltpu.sync_copy(dst_hbm.at[pl.ds(8 * m, nrow)],
                            dstb.at[pl.ds(0, nrow)])
            pltpu.sync_copy(ew_hbm.at[pl.ds(8 * m, nrow)],
                            ewb.at[pl.ds(0, nrow)])
            descs = [pltpu.async_copy(ewb.at[j], acc.at[dstb.at[j]], sem,
                                      add=True) for j in range(nrow)]
            for dsc in descs:
                dsc.wait()

        def rnd(t, carry):
            do_block(wid + NW * t, 8)
            return carry

        lax.fori_loop(0, DEG_FULL // NW, rnd, 0)

        @pl.when(wid < DEG_FULL - (DEG_FULL // NW) * NW)
        def _():
            do_block((DEG_FULL // NW) * NW + wid, 8)

        @pl.when(wid == NW - 1)
        def _():
            do_block(DEG_FULL, DEG_ROWS - DEG_FULL * 8)

        plsc.subcore_barrier()

        @pl.when(s < 15)
        def _():
            pltpu.sync_copy(acc.at[pl.ds(s * 640, 640)], zb)
            pltpu.sync_copy(zb, out_hbm.at[pl.ds(c * N + s * 640, 640)])

        @pl.when(s == 15)
        def _():
            pltpu.sync_copy(acc.at[pl.ds(9600, 400)], zb.at[pl.ds(0, 400)])
            pltpu.sync_copy(zb.at[pl.ds(0, 400)],
                            out_hbm.at[pl.ds(c * N + 9600, 400)])

    return deg_kernel(dst2, ew2)


ECH = 80
NCH_TOT = E // ECH
TPW = NCH_TOT // NW
TRIPLES = TPW // 3
AZR = 80


def _sc_aggregate(hp, src, dst, ewrf):
    bufspec = [
        pltpu.VMEM((ECH,), _i32),
        pltpu.VMEM((ECH,), _i32),
        pltpu.VMEM((ECH * 16,), _f32),
        pltpu.VMEM((ECH, D), _f32),
    ]

    @functools.partial(
        pl.kernel,
        out_type=jax.ShapeDtypeStruct((NC * N, D), _f32),
        mesh=_mesh(),
        scratch_types=bufspec * 3 + [
            pltpu.VMEM_SHARED((N, D), _f32),
            pltpu.SemaphoreType.DMA,
            pltpu.SemaphoreType.DMA,
            pltpu.SemaphoreType.DMA,
        ],
    )
    def agg_kernel(hp_hbm, src_hbm, dst_hbm, ewr_hbm, out_hbm,
                   srcA, dstA, ewrA, rowsA, srcB, dstB, ewrB, rowsB,
                   srcC, dstC, ewrC, rowsC, acc, sem_g, sem_s, sem_i):
        c = lax.axis_index("c")
        s = lax.axis_index("s")
        wid = s * NC + c
        zeros16 = jnp.zeros((16,), _f32)
        BUFS = ((srcA, dstA, ewrA, rowsA),
                (srcB, dstB, ewrB, rowsB),
                (srcC, dstC, ewrC, rowsC))

        def zrow(i, carry):
            for j in range(D // 16):
                rowsA[i, pl.ds(j * 16, 16)] = zeros16
            return carry

        lax.fori_loop(0, AZR, zrow, 0)

        @pl.when(s < 15)
        def _():
            for k in range(RPT // AZR):
                pltpu.sync_copy(rowsA, acc.at[pl.ds(s * RPT + k * AZR, AZR)])

        @pl.when(s == 15)
        def _():
            for k in range(5):
                pltpu.sync_copy(rowsA, acc.at[pl.ds(9600 + k * AZR, AZR)])

        plsc.subcore_barrier()

        def base(t):
            return (wid + NW * t) * ECH

        def issue_idx(t, bs):
            pltpu.async_copy(src_hbm.at[pl.ds(base(t), ECH)], bs[0], sem_i)
            pltpu.async_copy(dst_hbm.at[pl.ds(base(t), ECH)], bs[1], sem_i)
            pltpu.async_copy(ewr_hbm.at[pl.ds(base(t) * 16, ECH * 16)],
                             bs[2], sem_i)

        def wait_idx(t, bs):
            pltpu.make_async_copy(src_hbm.at[pl.ds(base(t), ECH)], bs[0],
                                  sem_i).wait()
            pltpu.make_async_copy(dst_hbm.at[pl.ds(base(t), ECH)], bs[1],
                                  sem_i).wait()
            pltpu.make_async_copy(ewr_hbm.at[pl.ds(base(t) * 16, ECH * 16)],
                                  bs[2], sem_i).wait()

        def issue_gather(t, bs):
            pltpu.async_copy(hp_hbm.at[bs[0]], bs[3], sem_g)

        def wait_gather(t, bs):
            pltpu.make_async_copy(hp_hbm.at[bs[0]], bs[3], sem_g).wait()

        def issue_scatter(bs):
            pltpu.async_copy(bs[3], acc.at[bs[1]], sem_s, add=True)

        def wait_scatter(bs):
            pltpu.make_async_copy(bs[3], acc.at[bs[1]], sem_s).wait()

        def scale(bs):
            ewr_v = bs[2]
            rows_v = bs[3]

            def scale_row(r, carry2):
                for l in range(8):
                    w = ewr_v[pl.ds((r * 8 + l) * 16, 16)]
                    e = r * 8 + l
                    for j in range(D // 16):
                        rows_v[e, pl.ds(j * 16, 16)] = (
                            rows_v[e, pl.ds(j * 16, 16)] * w)
                return carry2

            lax.fori_loop(0, ECH // 8, scale_row, 0)

        def step(t, cur, nxt, guard_scatter_wait, k=None):
            if guard_scatter_wait is None:
                wait_scatter(nxt)
            elif guard_scatter_wait:
                @pl.when(k >= 1)
                def _():
                    wait_scatter(nxt)

            issue_idx(t + 1, nxt)
            wait_idx(t + 1, nxt)
            issue_gather(t + 1, nxt)
            wait_gather(t, cur)
            scale(cur)
            issue_scatter(cur)

        issue_idx(0, BUFS[0])
        wait_idx(0, BUFS[0])
        issue_gather(0, BUFS[0])

        def body(k, carry):
            t0 = 3 * k
            step(t0, BUFS[0], BUFS[1], True, k)
            step(t0 + 1, BUFS[1], BUFS[2], True, k)
            step(t0 + 2, BUFS[2], BUFS[0], None)
            return carry

        lax.fori_loop(0, TRIPLES, body, 0)
        t1 = 3 * TRIPLES
        wait_scatter(BUFS[1])
        issue_idx(t1 + 1, BUFS[1])
        wait_idx(t1 + 1, BUFS[1])
        issue_gather(t1 + 1, BUFS[1])
        wait_gather(t1, BUFS[0])
        scale(BUFS[0])
        issue_scatter(BUFS[0])
        wait_gather(t1 + 1, BUFS[1])
        scale(BUFS[1])
        issue_scatter(BUFS[1])
        wait_scatter(BUFS[2])
        wait_scatter(BUFS[0])
        wait_scatter(BUFS[1])
        plsc.subcore_barrier()

        @pl.when(s < 15)
        def _():
            for k in range(RPT // AZR):
                pltpu.sync_copy(acc.at[pl.ds(s * RPT + k * AZR, AZR)], rowsA)
                pltpu.sync_copy(rowsA,
                                out_hbm.at[pl.ds(c * N + s * RPT + k * AZR, AZR)])

        @pl.when(s == 15)
        def _():
            for k in range(5):
                pltpu.sync_copy(acc.at[pl.ds(9600 + k * AZR, AZR)], rowsA)
                pltpu.sync_copy(rowsA,
                                out_hbm.at[pl.ds(c * N + 9600 + k * AZR, AZR)])

    return agg_kernel(hp, src, dst, ewrf)


def _tc_pre(x, W1, deg2, ew2):
    def body(x_ref, w_ref, deg_ref, ew_ref, hp_ref, dis_ref, ewr_ref):
        deg = deg_ref[0] + deg_ref[1] + 1.0
        dis = jnp.where(deg > 0, lax.rsqrt(jnp.maximum(deg, 1e-12)), 0.0)
        h = jnp.dot(x_ref[...], w_ref[...], preferred_element_type=_f32)
        hp_ref[...] = h * dis[:, None]
        dis_ref[...] = dis[:, None]
        col = lax.broadcasted_iota(_i32, (8, D), 1) // 16
        row = lax.broadcasted_iota(_i32, (8, D), 0)
        rep = jnp.where(col == row, 1.0, 0.0).astype(_f32)
        ewr_ref[...] = jnp.dot(ew_ref[...], rep, preferred_element_type=_f32)

    return pl.pallas_call(
        body,
        out_shape=(
            jax.ShapeDtypeStruct((N, D), _f32),
            jax.ShapeDtypeStruct((N, 1), _f32),
            jax.ShapeDtypeStruct((E // 8, D), _f32),
        ),
    )(x, W1, deg2, ew2)


def _tc_mid(s2, hp, dis, b, Wn):
    def body(s_ref, hp_ref, dis_ref, b_ref, w_ref, out_ref):
        t = (s_ref[0] + s_ref[1] + hp_ref[...]) * dis_ref[...] + b_ref[...]
        a = jnp.maximum(t, 0.0)
        h = jnp.dot(a, w_ref[...], preferred_element_type=_f32)
        out_ref[...] = h * dis_ref[...]

    return pl.pallas_call(
        body,
        out_shape=jax.ShapeDtypeStruct((N, D), _f32),
    )(s2, hp, dis, b, Wn)


def _tc_fin(s2, hp, dis, b):
    def body(s_ref, hp_ref, dis_ref, b_ref, out_ref):
        out_ref[...] = (s_ref[0] + s_ref[1] + hp_ref[...]) * dis_ref[...] + b_ref[...]

    return pl.pallas_call(
        body,
        out_shape=jax.ShapeDtypeStruct((N, D), _f32),
    )(s2, hp, dis, b)


def kernel(x, edge_index, edge_weight, W1, b1, W2, b2, W3, b3):
    ei = edge_index
    ew = edge_weight.astype(_f32)

    deg2 = _sc_degree(ei[1].reshape(DEG_ROWS, 128),
                      ew.reshape(DEG_ROWS, 128)).reshape(NC, N)
    hp1, dis, ewr = _tc_pre(x, W1, deg2, ew.reshape(E // 8, 8))
    ewrf = ewr.reshape(E * 16)
    src = ei[0]
    dst = ei[1]
    s1 = _sc_aggregate(hp1, src, dst, ewrf).reshape(NC, N, D)
    hp2 = _tc_mid(s1, hp1, dis, b1, W2)
    s2 = _sc_aggregate(hp2, src, dst, ewrf).reshape(NC, N, D)
    hp3 = _tc_mid(s2, hp2, dis, b2, W3)
    s3 = _sc_aggregate(hp3, src, dst, ewrf).reshape(NC, N, D)
    return _tc_fin(s3, hp3, dis, b3)
